# Initial kernel scaffold; baseline (speedup 1.0000x reference)
import jax
import jax.numpy as jnp
from jax.experimental import pallas as pl


def kernel(x, edge_index, edge_attr, W1q, b1q, W1k, b1k, W1v, b1v, W1e, W1s, b1s, W2q, b2q, W2k, b2k, W2v, b2v, W2e, W2s, b2s):
    raise NotImplementedError("write your pallas kernel here")


# speedup vs baseline: 1.6492x; 1.6492x over previous
import functools
import jax
import jax.numpy as jnp
from jax import lax
from jax.experimental import pallas as pl
from jax.experimental.pallas import tpu as pltpu

_N = 10000
_E = 160000
_D = 256
_DE = 16
_ROWS = 1000


def _proj_body(x_ref, wcat_ref, bcat_ref, wepad_ref, cat_ref, qe_ref):
    x = x_ref[...]
    cat = jnp.dot(x, wcat_ref[...], preferred_element_type=jnp.float32) + bcat_ref[...]
    cat_ref[...] = cat
    q = cat[:, :_D]
    qe_ref[...] = jnp.dot(q, wepad_ref[...], preferred_element_type=jnp.float32)


def _tc_proj(x, wcat, bcat, wepad):
    grid = (_N // _ROWS,)
    return pl.pallas_call(
        _proj_body,
        grid=grid,
        in_specs=[
            pl.BlockSpec((_ROWS, _D), lambda i: (i, 0)),
            pl.BlockSpec((_D, 4 * _D), lambda i: (0, 0)),
            pl.BlockSpec((1, 4 * _D), lambda i: (0, 0)),
            pl.BlockSpec((_D, 128), lambda i: (0, 0)),
        ],
        out_specs=[
            pl.BlockSpec((_ROWS, 4 * _D), lambda i: (i, 0)),
            pl.BlockSpec((_ROWS, 128), lambda i: (i, 0)),
        ],
        out_shape=[
            jax.ShapeDtypeStruct((_N, 4 * _D), jnp.float32),
            jax.ShapeDtypeStruct((_N, 128), jnp.float32),
        ],
    )(x, wcat, bcat, wepad)


def _fin_body(numer_ref, aux_ref, skip_ref, wet_ref, out_ref, *, relu):
    aux = aux_ref[...]
    r = 1.0 / (aux[:, 16:17] + 1e-16)
    out = (
        numer_ref[...] * r
        + jnp.dot(aux * r, wet_ref[...], preferred_element_type=jnp.float32)
        + skip_ref[...]
    )
    if relu:
        out = jnp.maximum(out, 0.0)
    out_ref[...] = out


def _tc_finish(numer, aux, skip, wet_pad, relu):
    grid = (_N // _ROWS,)
    return pl.pallas_call(
        functools.partial(_fin_body, relu=relu),
        grid=grid,
        in_specs=[
            pl.BlockSpec((_ROWS, _D), lambda i: (i, 0)),
            pl.BlockSpec((_ROWS, 128), lambda i: (i, 0)),
            pl.BlockSpec((_ROWS, _D), lambda i: (i, 0)),
            pl.BlockSpec((128, _D), lambda i: (0, 0)),
        ],
        out_specs=pl.BlockSpec((_ROWS, _D), lambda i: (i, 0)),
        out_shape=jax.ShapeDtypeStruct((_N, _D), jnp.float32),
    )(numer, aux, skip, wet_pad)


def _edge_phase(q, k, v, qe, src, dst, edge_attr):
    alpha = (
        jnp.sum(q[dst] * k[src], axis=-1) + jnp.sum(qe[dst] * edge_attr, axis=-1)
    ) * (1.0 / 16.0)
    ex = jnp.exp(alpha)
    den = jax.ops.segment_sum(ex, dst, num_segments=_N)
    numer = jax.ops.segment_sum(ex[:, None] * v[src], dst, num_segments=_N)
    wnum = jax.ops.segment_sum(ex[:, None] * edge_attr, dst, num_segments=_N)
    aux = jnp.zeros((_N, 128), jnp.float32)
    aux = aux.at[:, :16].set(wnum)
    aux = aux.at[:, 16:32].set(den[:, None])
    return numer, aux


def _layer(x, src, dst, edge_attr, Wq, bq, Wk, bk, Wv, bv, We, Ws, bs, relu):
    wcat = jnp.concatenate([Wq.T, Wk.T, Wv.T, Ws.T], axis=1)
    bcat = jnp.concatenate([bq, bk, bv, bs])[None, :]
    wepad = jnp.zeros((_D, 128), jnp.float32).at[:, :_DE].set(We)
    cat, qe = _tc_proj(x, wcat, bcat, wepad)
    q = cat[:, :_D]
    k = cat[:, _D:2 * _D]
    v = cat[:, 2 * _D:3 * _D]
    skip = cat[:, 3 * _D:]
    numer, aux = _edge_phase(q, k, v, qe[:, :_DE], src, dst, edge_attr)
    wet_pad = jnp.zeros((128, _D), jnp.float32).at[:_DE, :].set(We.T)
    return _tc_finish(numer, aux, skip, wet_pad, relu)


def kernel(x, edge_index, edge_attr,
           W1q, b1q, W1k, b1k, W1v, b1v, W1e, W1s, b1s,
           W2q, b2q, W2k, b2k, W2v, b2v, W2e, W2s, b2s):
    src = edge_index[0]
    dst = edge_index[1]
    h = _layer(x, src, dst, edge_attr,
               W1q, b1q, W1k, b1k, W1v, b1v, W1e, W1s, b1s, relu=True)
    out = _layer(h, src, dst, edge_attr,
                 W2q, b2q, W2k, b2k, W2v, b2v, W2e, W2s, b2s, relu=False)
    return out


# speedup vs baseline: 3.3189x; 2.0124x over previous
import functools
import jax
import jax.numpy as jnp
from jax import lax
from jax.experimental import pallas as pl
from jax.experimental.pallas import tpu as pltpu
from jax.experimental.pallas import tpu_sc as plsc

_N = 10000
_E = 160000
_D = 256
_DE = 16
_ROWS = 1000

_NC = 2
_NS = 16
_NW = _NC * _NS
_C = 128
_NCHUNK = _E // _C

_sc_mesh = plsc.VectorSubcoreMesh(
    core_axis_name="c", subcore_axis_name="s", num_cores=_NC, num_subcores=_NS
)

_GD = lax.GatherDimensionNumbers(
    offset_dims=(), collapsed_slice_dims=(0,), start_index_map=(0,)
)


def _lane_perm(v, idx):
    return lax.gather(v, idx[:, None], _GD, slice_sizes=(1,),
                      mode=lax.GatherScatterMode.PROMISE_IN_BOUNDS)


_STRIPE = 624


def _per_stripe(sid, body):
    @pl.when(sid < _NS - 1)
    def _():
        body(pl.ds(pl.multiple_of(sid * _STRIPE, 8), _STRIPE))

    @pl.when(sid == _NS - 1)
    def _():
        body(pl.ds((_NS - 1) * _STRIPE, _N - (_NS - 1) * _STRIPE))


@functools.partial(
    pl.kernel,
    out_type=jax.ShapeDtypeStruct((_E,), jnp.float32),
    mesh=_sc_mesh,
    scratch_types=[
        pltpu.VMEM((_C,), jnp.int32),
        pltpu.VMEM((_C,), jnp.int32),
        pltpu.VMEM((_C, _D + 128), jnp.float32),
        pltpu.VMEM((_C, _D), jnp.float32),
        pltpu.VMEM((_C, _DE), jnp.float32),
        pltpu.VMEM((_C,), jnp.float32),
        pltpu.SemaphoreType.DMA,
    ],
)
def _sc_alpha(qcat_hbm, k_hbm, ea_hbm, src_hbm, dst_hbm,
              ex_hbm,
              dstv, srcv, qrows, krows, eav, exbuf, sem):
    cid = lax.axis_index("c")
    sid = lax.axis_index("s")
    wid = sid * _NC + cid
    nch = (_NCHUNK - wid + _NW - 1) // _NW
    lanes = lax.iota(jnp.int32, 16)

    def chunk_body(t, _):
        base = (t * _NW + wid) * _C
        pltpu.sync_copy(dst_hbm.at[pl.ds(base, _C)], dstv)
        pltpu.sync_copy(src_hbm.at[pl.ds(base, _C)], srcv)
        pltpu.sync_copy(ea_hbm.at[pl.ds(base, _C)], eav)
        pltpu.async_copy(qcat_hbm.at[dstv], qrows, sem).wait()
        pltpu.async_copy(k_hbm.at[srcv], krows, sem).wait()

        def grp(g, _):
            res = jnp.zeros((16,), jnp.float32)
            for j in range(16):
                e = g * 16 + j
                acc = qrows[e, pl.ds(_D, 16)] * eav[e, :]
                for dd in range(_D // 16):
                    acc = acc + qrows[e, pl.ds(dd * 16, 16)] * krows[e, pl.ds(dd * 16, 16)]
                for sh in (8, 4, 2, 1):
                    acc = acc + _lane_perm(acc, lanes ^ sh)
                res = jnp.where(lanes == j, acc, res)
            exbuf[pl.ds(g * 16, 16)] = jnp.exp(res * 0.0625)
            return 0

        lax.fori_loop(0, _C // 16, grp, 0, unroll=False)
        pltpu.sync_copy(exbuf, ex_hbm.at[pl.ds(base, _C)])
        return 0

    lax.fori_loop(0, nch, chunk_body, 0, unroll=False)


@functools.partial(
    pl.kernel,
    out_type=[
        jax.ShapeDtypeStruct((_N, 128), jnp.float32),
        jax.ShapeDtypeStruct((_N, 128), jnp.float32),
    ],
    mesh=_sc_mesh,
    scratch_types=[
        pltpu.VMEM((_C,), jnp.int32),
        pltpu.VMEM((_C,), jnp.float32),
        pltpu.VMEM((_C, _DE), jnp.float32),
        pltpu.VMEM((_C, 128), jnp.float32),
        pltpu.VMEM_SHARED((_N, 128), jnp.float32),
        pltpu.SemaphoreType.DMA,
    ],
)
def _sc_aux(ex_hbm, ea_hbm, dst_hbm, z128_hbm,
            aux0_hbm, aux1_hbm,
            dstv, exv, eav, auxmsg, aux_acc, sem):
    cid = lax.axis_index("c")
    sid = lax.axis_index("s")
    wid = sid * _NC + cid
    nch = (_NCHUNK - wid + _NW - 1) // _NW
    lanes = lax.iota(jnp.int32, 16)

    _per_stripe(sid, lambda sl: pltpu.sync_copy(z128_hbm.at[sl], aux_acc.at[sl]))

    def zrow(e, _):
        zero = jnp.zeros((16,), jnp.float32)
        for c in range(2, 8):
            auxmsg[e, pl.ds(c * 16, 16)] = zero
        return 0

    lax.fori_loop(0, _C, zrow, 0, unroll=False)
    plsc.subcore_barrier()

    def chunk_body(t, _):
        base = (t * _NW + wid) * _C
        pltpu.sync_copy(dst_hbm.at[pl.ds(base, _C)], dstv)
        pltpu.sync_copy(ex_hbm.at[pl.ds(base, _C)], exv)
        pltpu.sync_copy(ea_hbm.at[pl.ds(base, _C)], eav)

        def grp(g, _):
            ex16 = exv[pl.ds(g * 16, 16)]
            for j in range(16):
                e = g * 16 + j
                splat = _lane_perm(ex16, jnp.full((16,), j, jnp.int32))
                auxmsg[e, pl.ds(0, 16)] = splat * eav[e, :]
                auxmsg[e, pl.ds(16, 16)] = jnp.where(lanes == 0, splat, 0.0)
            return 0

        lax.fori_loop(0, _C // 16, grp, 0, unroll=False)
        pltpu.sync_copy(auxmsg, aux_acc.at[dstv], add=True)
        return 0

    lax.fori_loop(0, nch, chunk_body, 0, unroll=False)
    plsc.subcore_barrier()

    def wb(sl):
        @pl.when(cid == 0)
        def _():
            pltpu.sync_copy(aux_acc.at[sl], aux0_hbm.at[sl])

        @pl.when(cid == 1)
        def _():
            pltpu.sync_copy(aux_acc.at[sl], aux1_hbm.at[sl])

    _per_stripe(sid, wb)


@functools.partial(
    pl.kernel,
    out_type=[
        jax.ShapeDtypeStruct((_N, 128), jnp.float32),
        jax.ShapeDtypeStruct((_N, 128), jnp.float32),
    ],
    mesh=_sc_mesh,
    scratch_types=[
        pltpu.VMEM((_C,), jnp.int32),
        pltpu.VMEM((_C,), jnp.int32),
        pltpu.VMEM((_C,), jnp.float32),
        pltpu.VMEM((_C, 128), jnp.float32),
        pltpu.VMEM((_C, 128), jnp.float32),
        pltpu.VMEM_SHARED((_N, 128), jnp.float32),
        pltpu.SemaphoreType.DMA,
    ],
)
def _sc_agg(v0_hbm, v1_hbm, ex_hbm, src_hbm, dst_hbm, z128_hbm,
            numer0_hbm, numer1_hbm,
            dstv, srcv, exv, vrows, msg, accum, sem):
    cid = lax.axis_index("c")
    sid = lax.axis_index("s")
    nch = (_NCHUNK - sid + _NS - 1) // _NS
    lanes = lax.iota(jnp.int32, 16)

    _per_stripe(sid, lambda sl: pltpu.sync_copy(z128_hbm.at[sl], accum.at[sl]))
    plsc.subcore_barrier()

    def chunk_body(t, _):
        base = (t * _NS + sid) * _C
        pltpu.sync_copy(dst_hbm.at[pl.ds(base, _C)], dstv)
        pltpu.sync_copy(src_hbm.at[pl.ds(base, _C)], srcv)
        pltpu.sync_copy(ex_hbm.at[pl.ds(base, _C)], exv)

        @pl.when(cid == 0)
        def _():
            pltpu.async_copy(v0_hbm.at[srcv], vrows, sem).wait()

        @pl.when(cid == 1)
        def _():
            pltpu.async_copy(v1_hbm.at[srcv], vrows, sem).wait()

        def grp(g, _):
            ex16 = exv[pl.ds(g * 16, 16)]
            for j in range(16):
                e = g * 16 + j
                splat = _lane_perm(ex16, jnp.full((16,), j, jnp.int32))
                for c in range(8):
                    msg[e, pl.ds(c * 16, 16)] = vrows[e, pl.ds(c * 16, 16)] * splat
            return 0

        lax.fori_loop(0, _C // 16, grp, 0, unroll=False)
        pltpu.sync_copy(msg, accum.at[dstv], add=True)
        return 0

    lax.fori_loop(0, nch, chunk_body, 0, unroll=False)
    plsc.subcore_barrier()

    def wb(sl):
        @pl.when(cid == 0)
        def _():
            pltpu.sync_copy(accum.at[sl], numer0_hbm.at[sl])

        @pl.when(cid == 1)
        def _():
            pltpu.sync_copy(accum.at[sl], numer1_hbm.at[sl])

    _per_stripe(sid, wb)


def _proj_body(x_ref, wcat_ref, bcat_ref, wepad_ref, qcat_ref, k_ref, v0_ref, v1_ref, skip_ref):
    x = x_ref[...]
    cat = jnp.dot(x, wcat_ref[...], preferred_element_type=jnp.float32) + bcat_ref[...]
    q = cat[:, :_D]
    qe = jnp.dot(q, wepad_ref[...], preferred_element_type=jnp.float32)
    qcat_ref[...] = jnp.concatenate([q, qe], axis=1)
    k_ref[...] = cat[:, _D:2 * _D]
    v0_ref[...] = cat[:, 2 * _D:2 * _D + 128]
    v1_ref[...] = cat[:, 2 * _D + 128:3 * _D]
    skip_ref[...] = cat[:, 3 * _D:]


def _tc_proj(x, wcat, bcat, wepad):
    grid = (_N // _ROWS,)
    return pl.pallas_call(
        _proj_body,
        grid=grid,
        in_specs=[
            pl.BlockSpec((_ROWS, _D), lambda i: (i, 0)),
            pl.BlockSpec((_D, 4 * _D), lambda i: (0, 0)),
            pl.BlockSpec((1, 4 * _D), lambda i: (0, 0)),
            pl.BlockSpec((_D, 128), lambda i: (0, 0)),
        ],
        out_specs=[
            pl.BlockSpec((_ROWS, _D + 128), lambda i: (i, 0)),
            pl.BlockSpec((_ROWS, _D), lambda i: (i, 0)),
            pl.BlockSpec((_ROWS, 128), lambda i: (i, 0)),
            pl.BlockSpec((_ROWS, 128), lambda i: (i, 0)),
            pl.BlockSpec((_ROWS, _D), lambda i: (i, 0)),
        ],
        out_shape=[
            jax.ShapeDtypeStruct((_N, _D + 128), jnp.float32),
            jax.ShapeDtypeStruct((_N, _D), jnp.float32),
            jax.ShapeDtypeStruct((_N, 128), jnp.float32),
            jax.ShapeDtypeStruct((_N, 128), jnp.float32),
            jax.ShapeDtypeStruct((_N, _D), jnp.float32),
        ],
    )(x, wcat, bcat, wepad)


def _fin_body(n0_ref, n1_ref, aux0_ref, aux1_ref, skip_ref, wet_ref, out_ref, *, relu):
    aux = aux0_ref[...] + aux1_ref[...]
    r = 1.0 / (aux[:, 16:17] + 1e-16)
    numer = jnp.concatenate([n0_ref[...], n1_ref[...]], axis=1)
    out = (
        numer * r
        + jnp.dot(aux * r, wet_ref[...], preferred_element_type=jnp.float32)
        + skip_ref[...]
    )
    if relu:
        out = jnp.maximum(out, 0.0)
    out_ref[...] = out


def _tc_finish(n0, n1, aux0, aux1, skip, wet_pad, relu):
    grid = (_N // _ROWS,)
    return pl.pallas_call(
        functools.partial(_fin_body, relu=relu),
        grid=grid,
        in_specs=[
            pl.BlockSpec((_ROWS, 128), lambda i: (i, 0)),
            pl.BlockSpec((_ROWS, 128), lambda i: (i, 0)),
            pl.BlockSpec((_ROWS, 128), lambda i: (i, 0)),
            pl.BlockSpec((_ROWS, 128), lambda i: (i, 0)),
            pl.BlockSpec((_ROWS, _D), lambda i: (i, 0)),
            pl.BlockSpec((128, _D), lambda i: (0, 0)),
        ],
        out_specs=pl.BlockSpec((_ROWS, _D), lambda i: (i, 0)),
        out_shape=jax.ShapeDtypeStruct((_N, _D), jnp.float32),
    )(n0, n1, aux0, aux1, skip, wet_pad)


def _layer(x, src, dst, edge_attr, zeros128, Wq, bq, Wk, bk, Wv, bv, We, Ws, bs, relu):
    wcat = jnp.concatenate([Wq.T, Wk.T, Wv.T, Ws.T], axis=1)
    bcat = jnp.concatenate([bq, bk, bv, bs])[None, :]
    wepad = jnp.zeros((_D, 128), jnp.float32).at[:, :_DE].set(We)
    qcat, k, v0, v1, skip = _tc_proj(x, wcat, bcat, wepad)
    ex = _sc_alpha(qcat, k, edge_attr, src, dst)
    aux0, aux1 = _sc_aux(ex, edge_attr, dst, zeros128)
    n0, n1 = _sc_agg(v0, v1, ex, src, dst, zeros128)
    wet_pad = jnp.zeros((128, _D), jnp.float32).at[:_DE, :].set(We.T)
    return _tc_finish(n0, n1, aux0, aux1, skip, wet_pad, relu)


def kernel(x, edge_index, edge_attr,
           W1q, b1q, W1k, b1k, W1v, b1v, W1e, W1s, b1s,
           W2q, b2q, W2k, b2k, W2v, b2v, W2e, W2s, b2s):
    src = edge_index[0]
    dst = edge_index[1]
    zeros128 = jnp.zeros((_N, 128), jnp.float32)
    h = _layer(x, src, dst, edge_attr, zeros128,
               W1q, b1q, W1k, b1k, W1v, b1v, W1e, W1s, b1s, relu=True)
    out = _layer(h, src, dst, edge_attr, zeros128,
                 W2q, b2q, W2k, b2k, W2v, b2v, W2e, W2s, b2s, relu=False)
    return out


# speedup vs baseline: 4.1921x; 1.2631x over previous
import functools
import jax
import jax.numpy as jnp
from jax import lax
from jax.experimental import pallas as pl
from jax.experimental.pallas import tpu as pltpu
from jax.experimental.pallas import tpu_sc as plsc

_N = 10000
_E = 160000
_D = 256
_DE = 16
_ROWS = 1000

_NC = 2
_NS = 16
_NW = _NC * _NS
_C = 128
_NCHUNK = _E // _C

_sc_mesh = plsc.VectorSubcoreMesh(
    core_axis_name="c", subcore_axis_name="s", num_cores=_NC, num_subcores=_NS
)

_GD = lax.GatherDimensionNumbers(
    offset_dims=(), collapsed_slice_dims=(0,), start_index_map=(0,)
)


def _lane_perm(v, idx):
    return lax.gather(v, idx[:, None], _GD, slice_sizes=(1,),
                      mode=lax.GatherScatterMode.PROMISE_IN_BOUNDS)


_STRIPE = 624


def _per_stripe(sid, body):
    @pl.when(sid < _NS - 1)
    def _():
        body(pl.ds(pl.multiple_of(sid * _STRIPE, 8), _STRIPE))

    @pl.when(sid == _NS - 1)
    def _():
        body(pl.ds((_NS - 1) * _STRIPE, _N - (_NS - 1) * _STRIPE))


_CA = 64
_NCHA = _E // _CA
_FULLA = _NCHA // _NW
_TAILA = _NCHA - _FULLA * _NW


@functools.partial(
    pl.kernel,
    out_type=jax.ShapeDtypeStruct((_E,), jnp.float32),
    mesh=_sc_mesh,
    scratch_types=[
        pltpu.VMEM((_CA,), jnp.int32), pltpu.VMEM((_CA,), jnp.int32),
        pltpu.VMEM((_CA,), jnp.int32), pltpu.VMEM((_CA,), jnp.int32),
        pltpu.VMEM((_CA, _DE), jnp.float32), pltpu.VMEM((_CA, _DE), jnp.float32),
        pltpu.VMEM((_CA, _D + 128), jnp.float32), pltpu.VMEM((_CA, _D + 128), jnp.float32),
        pltpu.VMEM((_CA, _D), jnp.float32), pltpu.VMEM((_CA, _D), jnp.float32),
        pltpu.VMEM((_CA,), jnp.float32),
        pltpu.SemaphoreType.DMA, pltpu.SemaphoreType.DMA,
    ],
)
def _sc_alpha(qcat_hbm, k_hbm, ea_hbm, src_hbm, dst_hbm,
              ex_hbm,
              dstv0, dstv1, srcv0, srcv1, eav0, eav1,
              qrows0, qrows1, krows0, krows1, exbuf, gsem0, gsem1):
    cid = lax.axis_index("c")
    sid = lax.axis_index("s")
    wid = sid * _NC + cid
    lanes = lax.iota(jnp.int32, 16)
    dstv = (dstv0, dstv1)
    srcv = (srcv0, srcv1)
    eav = (eav0, eav1)
    qrows = (qrows0, qrows1)
    krows = (krows0, krows1)
    gsem = (gsem0, gsem1)

    def fire(chunk_id, b):
        base = chunk_id * _CA
        pltpu.sync_copy(dst_hbm.at[pl.ds(base, _CA)], dstv[b])
        pltpu.sync_copy(src_hbm.at[pl.ds(base, _CA)], srcv[b])
        pltpu.sync_copy(ea_hbm.at[pl.ds(base, _CA)], eav[b])
        pltpu.async_copy(qcat_hbm.at[dstv[b]], qrows[b], gsem[b])
        pltpu.async_copy(k_hbm.at[srcv[b]], krows[b], gsem[b])

    def wait_g(b):
        pltpu.make_async_copy(qcat_hbm.at[dstv[b]], qrows[b], gsem[b]).wait()
        pltpu.make_async_copy(k_hbm.at[srcv[b]], krows[b], gsem[b]).wait()

    def compute(chunk_id, b):
        qr, kr, ea = qrows[b], krows[b], eav[b]

        def grp(g, _):
            res = jnp.zeros((16,), jnp.float32)
            for j in range(16):
                e = g * 16 + j
                terms = [qr[e, pl.ds(_D, 16)] * ea[e, :]]
                for dd in range(_D // 16):
                    terms.append(qr[e, pl.ds(dd * 16, 16)] * kr[e, pl.ds(dd * 16, 16)])
                while len(terms) > 1:
                    terms = [a + b2 for a, b2 in zip(terms[::2], terms[1::2])] +                             (terms[-1:] if len(terms) % 2 else [])
                acc = terms[0]
                for sh in (8, 4, 2, 1):
                    acc = acc + _lane_perm(acc, lanes ^ sh)
                res = jnp.where(lanes == j, acc, res)
            exbuf[pl.ds(g * 16, 16)] = jnp.exp(res * 0.0625)
            return 0

        lax.fori_loop(0, _CA // 16, grp, 0, unroll=False)
        pltpu.sync_copy(exbuf, ex_hbm.at[pl.ds(chunk_id * _CA, _CA)])

    def cidx(t):
        return t * _NW + wid

    fire(cidx(0), 0)

    def pair_body(pr, _):
        t0 = pr * 2
        fire(cidx(t0 + 1), 1)
        wait_g(0)
        compute(cidx(t0), 0)

        @pl.when(t0 + 2 < _FULLA)
        def _():
            fire(cidx(t0 + 2), 0)

        wait_g(1)
        compute(cidx(t0 + 1), 1)
        return 0

    lax.fori_loop(0, _FULLA // 2, pair_body, 0, unroll=False)

    @pl.when(wid < _TAILA)
    def _():
        tail_id = _FULLA * _NW + wid
        fire(tail_id, 0)
        wait_g(0)
        compute(tail_id, 0)


@functools.partial(
    pl.kernel,
    out_type=[
        jax.ShapeDtypeStruct((_N, 128), jnp.float32),
        jax.ShapeDtypeStruct((_N, 128), jnp.float32),
    ],
    mesh=_sc_mesh,
    scratch_types=[
        pltpu.VMEM((_C,), jnp.int32),
        pltpu.VMEM((_C,), jnp.float32),
        pltpu.VMEM((_C, _DE), jnp.float32),
        pltpu.VMEM((_C, 128), jnp.float32),
        pltpu.VMEM_SHARED((_N, 128), jnp.float32),
        pltpu.SemaphoreType.DMA,
    ],
)
def _sc_aux(ex_hbm, ea_hbm, dst_hbm, z128_hbm,
            aux0_hbm, aux1_hbm,
            dstv, exv, eav, auxmsg, aux_acc, sem):
    cid = lax.axis_index("c")
    sid = lax.axis_index("s")
    wid = sid * _NC + cid
    nch = (_NCHUNK - wid + _NW - 1) // _NW
    lanes = lax.iota(jnp.int32, 16)

    _per_stripe(sid, lambda sl: pltpu.sync_copy(z128_hbm.at[sl], aux_acc.at[sl]))

    def zrow(e, _):
        zero = jnp.zeros((16,), jnp.float32)
        for c in range(2, 8):
            auxmsg[e, pl.ds(c * 16, 16)] = zero
        return 0

    lax.fori_loop(0, _C, zrow, 0, unroll=False)
    plsc.subcore_barrier()

    def chunk_body(t, _):
        base = (t * _NW + wid) * _C
        pltpu.sync_copy(dst_hbm.at[pl.ds(base, _C)], dstv)
        pltpu.sync_copy(ex_hbm.at[pl.ds(base, _C)], exv)
        pltpu.sync_copy(ea_hbm.at[pl.ds(base, _C)], eav)

        def grp(g, _):
            ex16 = exv[pl.ds(g * 16, 16)]
            for j in range(16):
                e = g * 16 + j
                splat = _lane_perm(ex16, jnp.full((16,), j, jnp.int32))
                auxmsg[e, pl.ds(0, 16)] = splat * eav[e, :]
                auxmsg[e, pl.ds(16, 16)] = jnp.where(lanes == 0, splat, 0.0)
            return 0

        lax.fori_loop(0, _C // 16, grp, 0, unroll=False)
        pltpu.sync_copy(auxmsg, aux_acc.at[dstv], add=True)
        return 0

    lax.fori_loop(0, nch, chunk_body, 0, unroll=False)
    plsc.subcore_barrier()

    def wb(sl):
        @pl.when(cid == 0)
        def _():
            pltpu.sync_copy(aux_acc.at[sl], aux0_hbm.at[sl])

        @pl.when(cid == 1)
        def _():
            pltpu.sync_copy(aux_acc.at[sl], aux1_hbm.at[sl])

    _per_stripe(sid, wb)


_CB = 80
_NCHB = _E // _CB


@functools.partial(
    pl.kernel,
    out_type=[
        jax.ShapeDtypeStruct((_N, 128), jnp.float32),
        jax.ShapeDtypeStruct((_N, 128), jnp.float32),
    ],
    mesh=_sc_mesh,
    scratch_types=[
        pltpu.VMEM((_CB,), jnp.int32), pltpu.VMEM((_CB,), jnp.int32),
        pltpu.VMEM((_CB,), jnp.int32), pltpu.VMEM((_CB,), jnp.int32),
        pltpu.VMEM((_CB,), jnp.float32), pltpu.VMEM((_CB,), jnp.float32),
        pltpu.VMEM((_CB, 128), jnp.float32), pltpu.VMEM((_CB, 128), jnp.float32),
        pltpu.VMEM_SHARED((_N, 128), jnp.float32),
        pltpu.SemaphoreType.DMA, pltpu.SemaphoreType.DMA,
    ],
)
def _sc_agg(v0_hbm, v1_hbm, ex_hbm, src_hbm, dst_hbm, z128_hbm,
            numer0_hbm, numer1_hbm,
            dstv0, dstv1, srcv0, srcv1, exv0, exv1, vrows0, vrows1,
            accum, gsem0, gsem1):
    cid = lax.axis_index("c")
    sid = lax.axis_index("s")
    lanes = lax.iota(jnp.int32, 16)
    dstv = (dstv0, dstv1)
    srcv = (srcv0, srcv1)
    exv = (exv0, exv1)
    vrows = (vrows0, vrows1)
    gsem = (gsem0, gsem1)

    _per_stripe(sid, lambda sl: pltpu.sync_copy(z128_hbm.at[sl], accum.at[sl]))
    plsc.subcore_barrier()

    def fire(t, b):
        base = (t * _NS + sid) * _CB
        pltpu.sync_copy(dst_hbm.at[pl.ds(base, _CB)], dstv[b])
        pltpu.sync_copy(src_hbm.at[pl.ds(base, _CB)], srcv[b])
        pltpu.sync_copy(ex_hbm.at[pl.ds(base, _CB)], exv[b])

        @pl.when(cid == 0)
        def _():
            pltpu.async_copy(v0_hbm.at[srcv[b]], vrows[b], gsem[b])

        @pl.when(cid == 1)
        def _():
            pltpu.async_copy(v1_hbm.at[srcv[b]], vrows[b], gsem[b])

    def wait_g(b):
        pltpu.make_async_copy(v0_hbm.at[srcv[b]], vrows[b], gsem[b]).wait()

    def compute(b):
        vr, ev = vrows[b], exv[b]

        def grp(g, _):
            ex16 = ev[pl.ds(g * 16, 16)]
            for j in range(16):
                e = g * 16 + j
                splat = _lane_perm(ex16, jnp.full((16,), j, jnp.int32))
                for c in range(8):
                    vr[e, pl.ds(c * 16, 16)] = vr[e, pl.ds(c * 16, 16)] * splat
            return 0

        lax.fori_loop(0, _CB // 16, grp, 0, unroll=False)
        pltpu.sync_copy(vr, accum.at[dstv[b]], add=True)

    npt = _NCHB // _NS
    fire(0, 0)

    def pair_body(pr, _):
        t0 = pr * 2
        fire(t0 + 1, 1)
        wait_g(0)
        compute(0)
        fire(t0 + 2, 0)
        wait_g(1)
        compute(1)
        return 0

    lax.fori_loop(0, (npt - 1) // 2, pair_body, 0, unroll=False)
    wait_g(0)
    compute(0)

    plsc.subcore_barrier()

    def wb(sl):
        @pl.when(cid == 0)
        def _():
            pltpu.sync_copy(accum.at[sl], numer0_hbm.at[sl])

        @pl.when(cid == 1)
        def _():
            pltpu.sync_copy(accum.at[sl], numer1_hbm.at[sl])

    _per_stripe(sid, wb)


def _proj_body(x_ref, wcat_ref, bcat_ref, wepad_ref, qcat_ref, k_ref, v0_ref, v1_ref, skip_ref):
    x = x_ref[...]
    cat = jnp.dot(x, wcat_ref[...], preferred_element_type=jnp.float32) + bcat_ref[...]
    q = cat[:, :_D]
    qe = jnp.dot(q, wepad_ref[...], preferred_element_type=jnp.float32)
    qcat_ref[...] = jnp.concatenate([q, qe], axis=1)
    k_ref[...] = cat[:, _D:2 * _D]
    v0_ref[...] = cat[:, 2 * _D:2 * _D + 128]
    v1_ref[...] = cat[:, 2 * _D + 128:3 * _D]
    skip_ref[...] = cat[:, 3 * _D:]


def _tc_proj(x, wcat, bcat, wepad):
    grid = (_N // _ROWS,)
    return pl.pallas_call(
        _proj_body,
        grid=grid,
        in_specs=[
            pl.BlockSpec((_ROWS, _D), lambda i: (i, 0)),
            pl.BlockSpec((_D, 4 * _D), lambda i: (0, 0)),
            pl.BlockSpec((1, 4 * _D), lambda i: (0, 0)),
            pl.BlockSpec((_D, 128), lambda i: (0, 0)),
        ],
        out_specs=[
            pl.BlockSpec((_ROWS, _D + 128), lambda i: (i, 0)),
            pl.BlockSpec((_ROWS, _D), lambda i: (i, 0)),
            pl.BlockSpec((_ROWS, 128), lambda i: (i, 0)),
            pl.BlockSpec((_ROWS, 128), lambda i: (i, 0)),
            pl.BlockSpec((_ROWS, _D), lambda i: (i, 0)),
        ],
        out_shape=[
            jax.ShapeDtypeStruct((_N, _D + 128), jnp.float32),
            jax.ShapeDtypeStruct((_N, _D), jnp.float32),
            jax.ShapeDtypeStruct((_N, 128), jnp.float32),
            jax.ShapeDtypeStruct((_N, 128), jnp.float32),
            jax.ShapeDtypeStruct((_N, _D), jnp.float32),
        ],
    )(x, wcat, bcat, wepad)


def _fin_body(n0_ref, n1_ref, aux0_ref, aux1_ref, skip_ref, wet_ref, out_ref, *, relu):
    aux = aux0_ref[...] + aux1_ref[...]
    r = 1.0 / (aux[:, 16:17] + 1e-16)
    numer = jnp.concatenate([n0_ref[...], n1_ref[...]], axis=1)
    out = (
        numer * r
        + jnp.dot(aux * r, wet_ref[...], preferred_element_type=jnp.float32)
        + skip_ref[...]
    )
    if relu:
        out = jnp.maximum(out, 0.0)
    out_ref[...] = out


def _tc_finish(n0, n1, aux0, aux1, skip, wet_pad, relu):
    grid = (_N // _ROWS,)
    return pl.pallas_call(
        functools.partial(_fin_body, relu=relu),
        grid=grid,
        in_specs=[
            pl.BlockSpec((_ROWS, 128), lambda i: (i, 0)),
            pl.BlockSpec((_ROWS, 128), lambda i: (i, 0)),
            pl.BlockSpec((_ROWS, 128), lambda i: (i, 0)),
            pl.BlockSpec((_ROWS, 128), lambda i: (i, 0)),
            pl.BlockSpec((_ROWS, _D), lambda i: (i, 0)),
            pl.BlockSpec((128, _D), lambda i: (0, 0)),
        ],
        out_specs=pl.BlockSpec((_ROWS, _D), lambda i: (i, 0)),
        out_shape=jax.ShapeDtypeStruct((_N, _D), jnp.float32),
    )(n0, n1, aux0, aux1, skip, wet_pad)


def _layer(x, src, dst, edge_attr, zeros128, Wq, bq, Wk, bk, Wv, bv, We, Ws, bs, relu):
    wcat = jnp.concatenate([Wq.T, Wk.T, Wv.T, Ws.T], axis=1)
    bcat = jnp.concatenate([bq, bk, bv, bs])[None, :]
    wepad = jnp.zeros((_D, 128), jnp.float32).at[:, :_DE].set(We)
    qcat, k, v0, v1, skip = _tc_proj(x, wcat, bcat, wepad)
    ex = _sc_alpha(qcat, k, edge_attr, src, dst)
    aux0, aux1 = _sc_aux(ex, edge_attr, dst, zeros128)
    n0, n1 = _sc_agg(v0, v1, ex, src, dst, zeros128)
    wet_pad = jnp.zeros((128, _D), jnp.float32).at[:_DE, :].set(We.T)
    return _tc_finish(n0, n1, aux0, aux1, skip, wet_pad, relu)


def kernel(x, edge_index, edge_attr,
           W1q, b1q, W1k, b1k, W1v, b1v, W1e, W1s, b1s,
           W2q, b2q, W2k, b2k, W2v, b2v, W2e, W2s, b2s):
    src = edge_index[0]
    dst = edge_index[1]
    zeros128 = jnp.zeros((_N, 128), jnp.float32)
    h = _layer(x, src, dst, edge_attr, zeros128,
               W1q, b1q, W1k, b1k, W1v, b1v, W1e, W1s, b1s, relu=True)
    out = _layer(h, src, dst, edge_attr, zeros128,
                 W2q, b2q, W2k, b2k, W2v, b2v, W2e, W2s, b2s, relu=False)
    return out
